# baseline (speedup 1.0000x reference)
import functools

import jax
import jax.numpy as jnp
from jax import lax
from jax.experimental import pallas as pl
from jax.experimental.pallas import tpu as pltpu
from jax.experimental.pallas import tpu_sc as plsc

L = 16
NBUF = 6
ROWCAP = 128
DUMMY = 1 << 14


@functools.cache
def _build(hidden: int, num_rows: int, batch: int):
    info = plsc.get_sparse_core_info()
    nc = info.num_cores
    nw = nc * info.num_subcores
    n_tc = -(-num_rows // 128)
    rng = -(-n_tc // nw)
    nq = hidden // L

    mesh = plsc.VectorSubcoreMesh(core_axis_name="c", subcore_axis_name="s")

    scratch = {
        "kl_v": pltpu.VMEM((batch + L,), jnp.int32),
        "vl_v": pltpu.VMEM((batch + L,), jnp.int32),
        "sv_v": pltpu.VMEM((batch + L,), jnp.int32),
        "hist": pltpu.VMEM((256,), jnp.int32),
        "claimed": pltpu.VMEM((256,), jnp.int32),
        "starts": pltpu.VMEM((256,), jnp.int32),
        "dlist": pltpu.VMEM((256,), jnp.int32),
        "tmp16": pltpu.VMEM((L,), jnp.int32),
        "rows_v": pltpu.VMEM((ROWCAP, 128), jnp.float32),
        "idxbuf": pltpu.VMEM((128,), jnp.int32),
        "tiles_v": [pltpu.VMEM((hidden, 128), jnp.float32) for _ in range(NBUF)],
        "sems": [pltpu.SemaphoreType.DMA for _ in range(NBUF)],
        "out_sem": pltpu.SemaphoreType.DMA,
    }

    @functools.partial(
        pl.kernel,
        mesh=mesh,
        out_type=jax.ShapeDtypeStruct((batch + 8, 128), jnp.float32),
        scratch_types=scratch,
        compiler_params=pltpu.CompilerParams(needs_layout_passes=False),
    )
    def gather_kernel(table_hbm, lab_hbm, out_hbm, kl_v, vl_v, sv_v, hist,
                      claimed, starts, dlist, tmp16, rows_v, idxbuf, tiles_v,
                      sems, out_sem):
        iota = lax.iota(jnp.int32, L)
        row_idx = [iota + q * L for q in range(nq)]
        wid = lax.axis_index("s") * nc + lax.axis_index("c")
        lo = wid * rng
        hi = jnp.minimum(lo + rng, n_tc)
        span = hi - lo

        for i in range(16):
            hist[pl.ds(i * L, L)] = jnp.zeros((L,), jnp.int32)
            claimed[pl.ds(i * L, L)] = jnp.zeros((L,), jnp.int32)

        pltpu.sync_copy(lab_hbm, sv_v.at[pl.ds(0, batch)])

        def p1(i, cnt):
            lvec = sv_v[pl.ds(i * L, L)]
            tc = lax.shift_right_logical(lvec, 7)
            inr = (tc >= lo) & (tc < hi)
            packed = ((lvec & 127) << 14) | (iota + i * L)
            plsc.store_compressed(kl_v.at[pl.ds(cnt, L)], tc - lo, mask=inr)
            plsc.store_compressed(vl_v.at[pl.ds(cnt, L)], packed, mask=inr)
            npop = plsc.all_reduce_population_count(inr)
            return cnt + npop[0]

        cnt = lax.fori_loop(0, batch // L, p1, jnp.int32(0))
        kl_v[pl.ds(cnt, L)] = jnp.full((L,), 255, jnp.int32)
        nwin = (cnt + L - 1) >> 4

        def run_geometry(ks, i):
            ksm1 = plsc.load_gather(kl_v, [i * L + jnp.maximum(iota - 1, 0)])
            head = (iota == 0) | (ks != ksm1)
            head_pos = plsc.cummax(jnp.where(head, iota, 0))
            rank = iota - head_pos
            tmp16[...] = jnp.where(head, iota, L)
            nxt = plsc.load_gather(tmp16, [jnp.minimum(iota + 1, L - 1)])
            nxt = jnp.where(iota < L - 1, nxt, L)
            nh = -jnp.flip(plsc.cummax(jnp.flip(-nxt)))
            return head, rank, nh - iota

        def p2(i, _):
            k = kl_v[pl.ds(i * L, L)]
            v = vl_v[pl.ds(i * L, L)]
            ks, vs = plsc.sort_key_val(k, v)
            kl_v[pl.ds(i * L, L)] = ks
            vl_v[pl.ds(i * L, L)] = vs
            head, _, runlen = run_geometry(ks, i)
            plsc.addupdate_scatter(hist, [ks], runlen, mask=head)
            return 0

        lax.fori_loop(0, nwin, p2, 0)

        def p3(i, carry):
            h = hist[pl.ds(i * L, L)]
            cs = plsc.cumsum(h)
            starts[pl.ds(i * L, L)] = cs - h + carry
            return carry + cs[L - 1]

        lax.fori_loop(0, 16, p3, jnp.int32(0))

        def p4(i, _):
            ks = kl_v[pl.ds(i * L, L)]
            vs = vl_v[pl.ds(i * L, L)]
            head, rank, runlen = run_geometry(ks, i)
            pos = (plsc.load_gather(starts, [ks])
                   + plsc.load_gather(claimed, [ks]) + rank)
            plsc.store_scatter(sv_v, [pos], vs)
            plsc.addupdate_scatter(claimed, [ks], runlen, mask=head)
            return 0

        lax.fori_loop(0, nwin, p4, 0)

        def p5(i, dcnt):
            h = hist[pl.ds(i * L, L)]
            slots = iota + i * L
            m = (h > 0) & (slots < span)
            plsc.store_compressed(dlist.at[pl.ds(dcnt, L)], slots, mask=m)
            npop = plsc.all_reduce_population_count(m)
            return dcnt + npop[0]

        dcnt = lax.fori_loop(0, 16, p5, jnp.int32(0))
        nwd = (dcnt + L - 1) >> 4

        def fire(slot_tc, buf):
            off = pl.multiple_of((lo + slot_tc) * 128, 128)
            pltpu.async_copy(
                table_hbm.at[:, pl.ds(off, 128)], tiles_v[buf], sems[buf]
            )

        def wait_tile(buf):
            pltpu.make_async_copy(
                table_hbm.at[:, pl.ds(0, 128)], tiles_v[buf], sems[buf]
            ).wait()

        def flush(k):
            for i in range(8):
                idv = idxbuf[pl.ds(i * L, L)]
                idxbuf[pl.ds(i * L, L)] = jnp.where(
                    i * L + iota < k, idv, batch + (iota & 7)
                )
            pltpu.async_copy(
                rows_v, out_hbm.at[idxbuf], out_sem
            ).wait()

        def p6(w, k):
            sl = dlist[pl.ds(w * L, L)] & 255
            stv = plsc.load_gather(starts, [sl])
            cnv = plsc.load_gather(hist, [sl])
            for j in range(NBUF):
                @pl.when(w * L + j < dcnt)
                def _():
                    fire(sl[j], j)

            def handle(j, k):
                n = jnp.where(w * L + j < dcnt, cnv[j], 0)
                st = stv[j]

                @pl.when(w * L + j < dcnt)
                def _():
                    wait_tile(j % NBUF)

                def batch16(m, k):
                    pk = plsc.load_gather(sv_v, [st + m * L + iota])
                    lanes = lax.shift_right_logical(pk, 14)
                    bs = pk & (DUMMY - 1)
                    vc = jnp.clip(n - m * L, 0, L)
                    for t in range(L):
                        @pl.when(t < vc)
                        def _():
                            pos = jnp.full((L,), k + t, jnp.int32)
                            col = jnp.full((L,), lanes[t], jnp.int32)
                            for q in range(nq):
                                vals = plsc.load_gather(
                                    tiles_v[j % NBUF], [row_idx[q], col]
                                )
                                plsc.store_scatter(
                                    rows_v, [pos, row_idx[q]], vals
                                )
                            plsc.store_scatter(
                                idxbuf,
                                [jnp.full((L,), k + t, jnp.int32)],
                                jnp.full((L,), bs[t], jnp.int32),
                                mask=iota == 0,
                            )
                    k = k + vc

                    @pl.when(k > 128 - L)
                    def _():
                        flush(k)

                    return jnp.where(k > 128 - L, 0, k)

                nb16 = (n + L - 1) >> 4
                k = lax.fori_loop(0, nb16, batch16, k)

                if j + NBUF < L:
                    @pl.when(w * L + j + NBUF < dcnt)
                    def _():
                        fire(sl[j + NBUF], (j + NBUF) % NBUF)

                return k

            for j in range(L):
                k = handle(j, k)
            return k

        k = lax.fori_loop(0, nwd, p6, jnp.int32(0))

        @pl.when(k > 0)
        def _():
            flush(k)

    return gather_kernel


def kernel(labels, embedding_table):
    num_rows, hidden = embedding_table.shape
    batch = labels.shape[0]
    gather_kernel = _build(hidden, num_rows, batch)
    out128 = gather_kernel(embedding_table.T, labels.astype(jnp.int32))
    return out128[:batch, :hidden]


# speedup vs baseline: 1.0144x; 1.0144x over previous
import functools

import jax
import jax.numpy as jnp
from jax import lax
from jax.experimental import pallas as pl
from jax.experimental.pallas import tpu as pltpu
from jax.experimental.pallas import tpu_sc as plsc

L = 16
NBUF = 7
ROWCAP = 128
DUMMY = 1 << 14


@functools.cache
def _build(hidden: int, num_rows: int, batch: int):
    info = plsc.get_sparse_core_info()
    nc = info.num_cores
    nw = nc * info.num_subcores
    n_tc = -(-num_rows // 128)
    rng = -(-n_tc // nw)
    nq = hidden // L

    mesh = plsc.VectorSubcoreMesh(core_axis_name="c", subcore_axis_name="s")

    scratch = {
        "kl_v": pltpu.VMEM((batch + L,), jnp.int32),
        "vl_v": pltpu.VMEM((batch + L,), jnp.int32),
        "sv_v": pltpu.VMEM((batch + L,), jnp.int32),
        "hist": pltpu.VMEM((256,), jnp.int32),
        "claimed": pltpu.VMEM((256,), jnp.int32),
        "starts": pltpu.VMEM((256,), jnp.int32),
        "dlist": pltpu.VMEM((256,), jnp.int32),
        "tmp16": pltpu.VMEM((L,), jnp.int32),
        "rows_v": pltpu.VMEM((ROWCAP, 128), jnp.float32),
        "idxbuf": pltpu.VMEM((128,), jnp.int32),
        "tiles_v": [pltpu.VMEM((hidden, 128), jnp.float32) for _ in range(NBUF)],
        "sems": [pltpu.SemaphoreType.DMA for _ in range(NBUF)],
        "out_sem": pltpu.SemaphoreType.DMA,
    }

    @functools.partial(
        pl.kernel,
        mesh=mesh,
        out_type=jax.ShapeDtypeStruct((batch + 8, 128), jnp.float32),
        scratch_types=scratch,
        compiler_params=pltpu.CompilerParams(needs_layout_passes=False),
    )
    def gather_kernel(table_hbm, lab_hbm, out_hbm, kl_v, vl_v, sv_v, hist,
                      claimed, starts, dlist, tmp16, rows_v, idxbuf, tiles_v,
                      sems, out_sem):
        iota = lax.iota(jnp.int32, L)
        row_idx = [iota + q * L for q in range(nq)]
        wid = lax.axis_index("s") * nc + lax.axis_index("c")
        lo = wid * rng
        hi = jnp.minimum(lo + rng, n_tc)
        span = hi - lo

        for i in range(16):
            hist[pl.ds(i * L, L)] = jnp.zeros((L,), jnp.int32)
            claimed[pl.ds(i * L, L)] = jnp.zeros((L,), jnp.int32)

        pltpu.sync_copy(lab_hbm, sv_v.at[pl.ds(0, batch)])

        UNROLL = 4

        def p1(i, cnt):
            win = []
            for u in range(UNROLL):
                w = i * UNROLL + u
                lvec = sv_v[pl.ds(w * L, L)]
                tc = lax.shift_right_logical(lvec, 7)
                inr = (tc >= lo) & (tc < hi)
                packed = ((lvec & 127) << 14) | (iota + w * L)
                win.append((tc - lo, packed, inr))
            for tcoff, packed, inr in win:
                plsc.store_compressed(kl_v.at[pl.ds(cnt, L)], tcoff, mask=inr)
                plsc.store_compressed(vl_v.at[pl.ds(cnt, L)], packed, mask=inr)
                cnt = cnt + plsc.all_reduce_population_count(inr)[0]
            return cnt

        cnt = lax.fori_loop(0, batch // (L * UNROLL), p1, jnp.int32(0))
        kl_v[pl.ds(cnt, L)] = jnp.full((L,), 255, jnp.int32)
        nwin = (cnt + L - 1) >> 4

        def run_geometry(ks, i):
            ksm1 = plsc.load_gather(kl_v, [i * L + jnp.maximum(iota - 1, 0)])
            head = (iota == 0) | (ks != ksm1)
            head_pos = plsc.cummax(jnp.where(head, iota, 0))
            rank = iota - head_pos
            tmp16[...] = jnp.where(head, iota, L)
            nxt = plsc.load_gather(tmp16, [jnp.minimum(iota + 1, L - 1)])
            nxt = jnp.where(iota < L - 1, nxt, L)
            nh = -jnp.flip(plsc.cummax(jnp.flip(-nxt)))
            return head, rank, nh - iota

        def p2(i, _):
            k = kl_v[pl.ds(i * L, L)]
            v = vl_v[pl.ds(i * L, L)]
            ks, vs = plsc.sort_key_val(k, v)
            kl_v[pl.ds(i * L, L)] = ks
            vl_v[pl.ds(i * L, L)] = vs
            head, _, runlen = run_geometry(ks, i)
            plsc.addupdate_scatter(hist, [ks], runlen, mask=head)
            return 0

        lax.fori_loop(0, nwin, p2, 0)

        def p3(i, carry):
            h = hist[pl.ds(i * L, L)]
            cs = plsc.cumsum(h)
            starts[pl.ds(i * L, L)] = cs - h + carry
            return carry + cs[L - 1]

        lax.fori_loop(0, 16, p3, jnp.int32(0))

        def p4(i, _):
            ks = kl_v[pl.ds(i * L, L)]
            vs = vl_v[pl.ds(i * L, L)]
            head, rank, runlen = run_geometry(ks, i)
            pos = (plsc.load_gather(starts, [ks])
                   + plsc.load_gather(claimed, [ks]) + rank)
            plsc.store_scatter(sv_v, [pos], vs)
            plsc.addupdate_scatter(claimed, [ks], runlen, mask=head)
            return 0

        lax.fori_loop(0, nwin, p4, 0)

        def p5(i, dcnt):
            h = hist[pl.ds(i * L, L)]
            slots = iota + i * L
            m = (h > 0) & (slots < span)
            plsc.store_compressed(dlist.at[pl.ds(dcnt, L)], slots, mask=m)
            npop = plsc.all_reduce_population_count(m)
            return dcnt + npop[0]

        dcnt = lax.fori_loop(0, 16, p5, jnp.int32(0))
        nwd = (dcnt + L - 1) >> 4

        def fire(slot_tc, buf):
            off = pl.multiple_of((lo + slot_tc) * 128, 128)
            pltpu.async_copy(
                table_hbm.at[:, pl.ds(off, 128)], tiles_v[buf], sems[buf]
            )

        def wait_tile(buf):
            pltpu.make_async_copy(
                table_hbm.at[:, pl.ds(0, 128)], tiles_v[buf], sems[buf]
            ).wait()

        def flush(k):
            for i in range(8):
                idv = idxbuf[pl.ds(i * L, L)]
                idxbuf[pl.ds(i * L, L)] = jnp.where(
                    i * L + iota < k, idv, batch + (iota & 7)
                )
            pltpu.async_copy(
                rows_v, out_hbm.at[idxbuf], out_sem
            ).wait()

        def p6(w, k):
            sl = dlist[pl.ds(w * L, L)] & 255
            stv = plsc.load_gather(starts, [sl])
            cnv = plsc.load_gather(hist, [sl])
            for j in range(NBUF):
                @pl.when(w * L + j < dcnt)
                def _():
                    fire(sl[j], j)

            def handle(j, k):
                n = jnp.where(w * L + j < dcnt, cnv[j], 0)
                st = stv[j]

                @pl.when(w * L + j < dcnt)
                def _():
                    wait_tile(j % NBUF)

                def batch16(m, k):
                    pk = plsc.load_gather(sv_v, [st + m * L + iota])
                    lanes = lax.shift_right_logical(pk, 14)
                    bs = pk & (DUMMY - 1)
                    vc = jnp.clip(n - m * L, 0, L)
                    for t in range(L):
                        @pl.when(t < vc)
                        def _():
                            pos = jnp.full((L,), k + t, jnp.int32)
                            col = jnp.full((L,), lanes[t], jnp.int32)
                            for q in range(nq):
                                vals = plsc.load_gather(
                                    tiles_v[j % NBUF], [row_idx[q], col]
                                )
                                plsc.store_scatter(
                                    rows_v, [pos, row_idx[q]], vals
                                )
                            plsc.store_scatter(
                                idxbuf,
                                [jnp.full((L,), k + t, jnp.int32)],
                                jnp.full((L,), bs[t], jnp.int32),
                                mask=iota == 0,
                            )
                    k = k + vc

                    @pl.when(k > 128 - L)
                    def _():
                        flush(k)

                    return jnp.where(k > 128 - L, 0, k)

                nb16 = (n + L - 1) >> 4
                k = lax.fori_loop(0, nb16, batch16, k)

                if j + NBUF < L:
                    @pl.when(w * L + j + NBUF < dcnt)
                    def _():
                        fire(sl[j + NBUF], (j + NBUF) % NBUF)

                return k

            for j in range(L):
                k = handle(j, k)
            return k

        k = lax.fori_loop(0, nwd, p6, jnp.int32(0))

        @pl.when(k > 0)
        def _():
            flush(k)

    return gather_kernel


def kernel(labels, embedding_table):
    num_rows, hidden = embedding_table.shape
    batch = labels.shape[0]
    gather_kernel = _build(hidden, num_rows, batch)
    out128 = gather_kernel(embedding_table.T, labels.astype(jnp.int32))
    return out128[:batch, :hidden]


# speedup vs baseline: 1.1216x; 1.1057x over previous
import functools

import jax
import jax.numpy as jnp
from jax import lax
from jax.experimental import pallas as pl
from jax.experimental.pallas import tpu as pltpu
from jax.experimental.pallas import tpu_sc as plsc

L = 16
NBUF = 8
WPF = 8
CHUNK = WPF * L


@functools.cache
def _build(hidden: int, num_rows: int, batch: int):
    info = plsc.get_sparse_core_info()
    nc = info.num_cores
    nw = nc * info.num_subcores
    b_per_w = batch // nw
    n_flush = b_per_w // CHUNK

    mesh = plsc.VectorSubcoreMesh(core_axis_name="c", subcore_axis_name="s")

    scratch = {
        "idx_v": pltpu.VMEM((b_per_w,), jnp.int32),
        "rows_v": [pltpu.VMEM((CHUNK, 128), jnp.float32) for _ in range(2)],
        "tiles_v": [pltpu.VMEM((hidden, 128), jnp.float32) for _ in range(NBUF)],
        "sems": [pltpu.SemaphoreType.DMA for _ in range(NBUF)],
        "out_sem": pltpu.SemaphoreType.DMA,
    }

    n_pair = n_flush // 2
    PAIR = 2 * CHUNK

    @functools.partial(
        pl.kernel,
        mesh=mesh,
        out_type=jax.ShapeDtypeStruct((batch, 128), jnp.float32),
        scratch_types=scratch,
        compiler_params=pltpu.CompilerParams(needs_layout_passes=False),
    )
    def gather_kernel(table_hbm, idx_hbm, out_hbm, idx_v, rows_v, tiles_v,
                      sems, out_sem):
        wid = lax.axis_index("s") * nc + lax.axis_index("c")
        base = wid * b_per_w
        pltpu.sync_copy(idx_hbm.at[pl.ds(base, b_per_w)], idx_v)

        row_idx = [lax.iota(jnp.int32, L) + q * L for q in range(hidden // L)]

        def fire(r, slot):
            off = pl.multiple_of((r >> 7) * 128, 128)
            return pltpu.async_copy(
                table_hbm.at[:, pl.ds(off, 128)], tiles_v[slot], sems[slot]
            )

        def extract(r, slot, buf, pos):
            col = jnp.full((L,), r & 127, jnp.int32)
            dst_row = jnp.full((L,), pos, jnp.int32)
            for q in range(hidden // L):
                vals = plsc.load_gather(tiles_v[slot], [row_idx[q], col])
                plsc.store_scatter(rows_v[buf], [dst_row, row_idx[q]], vals)

        def drain_out(buf):
            pltpu.make_async_copy(
                out_hbm.at[pl.ds(0, CHUNK)], rows_v[buf], out_sem
            ).wait()

        vec0 = idx_v[pl.ds(0, L)]
        prime = [fire(vec0[j], j) for j in range(NBUF)]
        for c in prime:
            del c

        def pair_body(g, _):
            gbase = g * PAIR
            vecs = [idx_v[pl.ds(gbase + w * L, L)] for w in range(PAIR // L)]
            la_off = jnp.minimum(gbase + PAIR, b_per_w - L)
            vecs.append(idx_v[pl.ds(la_off, L)])
            rs = [vecs[w][j] for w in range(len(vecs)) for j in range(L)]
            for buf in range(2):
                @pl.when(g > 0)
                def _():
                    drain_out(buf)
                for k in range(CHUNK):
                    kk = buf * CHUNK + k
                    slot = kk % NBUF
                    pltpu.make_async_copy(
                        table_hbm.at[:, pl.ds(0, 128)], tiles_v[slot],
                        sems[slot],
                    ).wait()
                    extract(rs[kk], slot, buf, k)
                    if kk + NBUF < PAIR:
                        fire(rs[kk + NBUF], slot)
                    else:
                        @pl.when(g < n_pair - 1)
                        def _():
                            fire(rs[kk + NBUF], slot)
                pltpu.async_copy(
                    rows_v[buf],
                    out_hbm.at[pl.ds(base + gbase + buf * CHUNK, CHUNK)],
                    out_sem,
                )
            return 0

        lax.fori_loop(0, n_pair, pair_body, 0)
        drain_out(0)
        drain_out(1)

    return gather_kernel


def kernel(labels, embedding_table):
    num_rows, hidden = embedding_table.shape
    batch = labels.shape[0]
    gather_kernel = _build(hidden, num_rows, batch)
    out128 = gather_kernel(embedding_table.T, labels.astype(jnp.int32))
    return out128[:, :hidden]


# speedup vs baseline: 1.1503x; 1.0256x over previous
import functools

import jax
import jax.numpy as jnp
from jax import lax
from jax.experimental import pallas as pl
from jax.experimental.pallas import tpu as pltpu
from jax.experimental.pallas import tpu_sc as plsc

L = 16
NBUF = 8
WPF = 4
CHUNK = WPF * L


@functools.cache
def _build(hidden: int, num_rows: int, batch: int):
    info = plsc.get_sparse_core_info()
    nc = info.num_cores
    nw = nc * info.num_subcores
    b_per_w = batch // nw
    n_flush = b_per_w // CHUNK

    mesh = plsc.VectorSubcoreMesh(core_axis_name="c", subcore_axis_name="s")

    scratch = {
        "idx_v": pltpu.VMEM((b_per_w,), jnp.int32),
        "rows_v": [pltpu.VMEM((CHUNK, 128), jnp.float32) for _ in range(2)],
        "tiles_v": [pltpu.VMEM((hidden, 128), jnp.float32) for _ in range(NBUF)],
        "sems": [pltpu.SemaphoreType.DMA for _ in range(NBUF)],
        "out_sem": pltpu.SemaphoreType.DMA,
    }

    n_pair = n_flush // 2
    PAIR = 2 * CHUNK

    @functools.partial(
        pl.kernel,
        mesh=mesh,
        out_type=jax.ShapeDtypeStruct((batch, 128), jnp.float32),
        scratch_types=scratch,
        compiler_params=pltpu.CompilerParams(needs_layout_passes=False),
    )
    def gather_kernel(table_hbm, idx_hbm, out_hbm, idx_v, rows_v, tiles_v,
                      sems, out_sem):
        wid = lax.axis_index("s") * nc + lax.axis_index("c")
        base = wid * b_per_w
        pltpu.sync_copy(idx_hbm.at[pl.ds(base, b_per_w)], idx_v)

        row_idx = [lax.iota(jnp.int32, L) + q * L for q in range(hidden // L)]

        def fire(r, slot):
            off = pl.multiple_of((r >> 7) * 128, 128)
            return pltpu.async_copy(
                table_hbm.at[:, pl.ds(off, 128)], tiles_v[slot], sems[slot]
            )

        def extract(r, slot, buf, pos):
            col = jnp.full((L,), r & 127, jnp.int32)
            dst_row = jnp.full((L,), pos, jnp.int32)
            for q in range(hidden // L):
                vals = plsc.load_gather(tiles_v[slot], [row_idx[q], col])
                plsc.store_scatter(rows_v[buf], [dst_row, row_idx[q]], vals)

        def drain_out(buf):
            pltpu.make_async_copy(
                table_hbm.at[:CHUNK, pl.ds(0, 128)], rows_v[buf], out_sem
            ).wait()

        vec0 = idx_v[pl.ds(0, L)]
        prime = [fire(vec0[j], j) for j in range(NBUF)]
        for c in prime:
            del c

        def pair_body(g, _):
            gbase = g * PAIR
            vecs = [idx_v[pl.ds(gbase + w * L, L)] for w in range(PAIR // L)]
            la_off = jnp.minimum(gbase + PAIR, b_per_w - L)
            vecs.append(idx_v[pl.ds(la_off, L)])
            rs = [vecs[w][j] for w in range(len(vecs)) for j in range(L)]
            for buf in range(2):
                @pl.when(g > 0)
                def _():
                    drain_out(buf)
                for k in range(CHUNK):
                    kk = buf * CHUNK + k
                    slot = kk % NBUF
                    pltpu.make_async_copy(
                        table_hbm.at[:, pl.ds(0, 128)], tiles_v[slot],
                        sems[slot],
                    ).wait()
                    extract(rs[kk], slot, buf, k)
                    if kk + NBUF < PAIR:
                        fire(rs[kk + NBUF], slot)
                    else:
                        @pl.when(g < n_pair - 1)
                        def _():
                            fire(rs[kk + NBUF], slot)
                pltpu.async_copy(
                    rows_v[buf],
                    out_hbm.at[pl.ds(base + gbase + buf * CHUNK, CHUNK)],
                    out_sem,
                )
            return 0

        lax.fori_loop(0, n_pair, pair_body, 0)
        drain_out(0)
        drain_out(1)

    return gather_kernel


def kernel(labels, embedding_table):
    num_rows, hidden = embedding_table.shape
    batch = labels.shape[0]
    gather_kernel = _build(hidden, num_rows, batch)
    out128 = gather_kernel(embedding_table.T, labels.astype(jnp.int32))
    return out128[:, :hidden]


# speedup vs baseline: 1.1533x; 1.0026x over previous
import functools

import jax
import jax.numpy as jnp
from jax import lax
from jax.experimental import pallas as pl
from jax.experimental.pallas import tpu as pltpu
from jax.experimental.pallas import tpu_sc as plsc

L = 16
NBUF = 8
WPF = 8
CHUNK = WPF * L


@functools.cache
def _build(hidden: int, num_rows: int, batch: int):
    info = plsc.get_sparse_core_info()
    nc = info.num_cores
    nw = nc * info.num_subcores
    b_per_w = batch // nw
    n_flush = b_per_w // CHUNK

    mesh = plsc.VectorSubcoreMesh(core_axis_name="c", subcore_axis_name="s")

    scratch = {
        "idx_v": pltpu.VMEM((b_per_w,), jnp.int32),
        "rows_v": [pltpu.VMEM((hidden, CHUNK), jnp.float32) for _ in range(2)],
        "tiles_v": [pltpu.VMEM((hidden, 128), jnp.float32) for _ in range(NBUF)],
        "sems": [pltpu.SemaphoreType.DMA for _ in range(NBUF)],
        "out_sem": pltpu.SemaphoreType.DMA,
    }

    n_pair = n_flush // 2
    PAIR = 2 * CHUNK

    @functools.partial(
        pl.kernel,
        mesh=mesh,
        out_type=jax.ShapeDtypeStruct((hidden, batch), jnp.float32),
        scratch_types=scratch,
        compiler_params=pltpu.CompilerParams(needs_layout_passes=False),
    )
    def gather_kernel(table_hbm, idx_hbm, out_hbm, idx_v, rows_v, tiles_v,
                      sems, out_sem):
        wid = lax.axis_index("s") * nc + lax.axis_index("c")
        base = wid * b_per_w
        pltpu.sync_copy(idx_hbm.at[pl.ds(base, b_per_w)], idx_v)

        row_idx = [lax.iota(jnp.int32, L) + q * L for q in range(hidden // L)]

        def fire(r, slot):
            off = pl.multiple_of((r >> 7) * 128, 128)
            return pltpu.async_copy(
                table_hbm.at[:, pl.ds(off, 128)], tiles_v[slot], sems[slot]
            )

        def extract(r, slot, buf, pos):
            col = jnp.full((L,), r & 127, jnp.int32)
            dst_col = jnp.full((L,), pos, jnp.int32)
            for q in range(hidden // L):
                vals = plsc.load_gather(tiles_v[slot], [row_idx[q], col])
                plsc.store_scatter(rows_v[buf], [row_idx[q], dst_col], vals)

        def drain_out(buf):
            pltpu.make_async_copy(
                table_hbm.at[:, pl.ds(0, CHUNK)], rows_v[buf], out_sem
            ).wait()

        vec0 = idx_v[pl.ds(0, L)]
        prime = [fire(vec0[j], j) for j in range(NBUF)]
        for c in prime:
            del c

        def pair_body(g, _):
            gbase = g * PAIR
            vecs = [idx_v[pl.ds(gbase + w * L, L)] for w in range(PAIR // L)]
            la_off = jnp.minimum(gbase + PAIR, b_per_w - L)
            vecs.append(idx_v[pl.ds(la_off, L)])
            rs = [vecs[w][j] for w in range(len(vecs)) for j in range(L)]
            for buf in range(2):
                @pl.when(g > 0)
                def _():
                    drain_out(buf)
                for k in range(CHUNK):
                    kk = buf * CHUNK + k
                    slot = kk % NBUF
                    pltpu.make_async_copy(
                        table_hbm.at[:, pl.ds(0, 128)], tiles_v[slot],
                        sems[slot],
                    ).wait()
                    extract(rs[kk], slot, buf, k)
                    if kk + NBUF < PAIR:
                        fire(rs[kk + NBUF], slot)
                    else:
                        @pl.when(g < n_pair - 1)
                        def _():
                            fire(rs[kk + NBUF], slot)
                pltpu.async_copy(
                    rows_v[buf],
                    out_hbm.at[:, pl.ds(base + gbase + buf * CHUNK, CHUNK)],
                    out_sem,
                )
            return 0

        lax.fori_loop(0, n_pair, pair_body, 0)
        drain_out(0)
        drain_out(1)

    return gather_kernel


def kernel(labels, embedding_table):
    num_rows, hidden = embedding_table.shape
    batch = labels.shape[0]
    gather_kernel = _build(hidden, num_rows, batch)
    out_t = gather_kernel(embedding_table.T, labels.astype(jnp.int32))
    return out_t.T
